# baseline (speedup 1.0000x reference)
import functools

import jax
import jax.numpy as jnp
from jax import lax
from jax.experimental import pallas as pl
from jax.experimental.pallas import tpu as pltpu
from jax.experimental.pallas import tpu_sc as plsc

_S = 2048
_H = 12
_DH = 64
_DM = 768
_NC = 2
_NS = 16
_NW = _NC * _NS
_NIDX = 2 * _S
_ROWS_PER_W = _NIDX // _NW
_T = 512
_NB = _S // _T
_SCALE = 1.0 / 8.0


def _proj3_body(qi_ref, ki_ref, vi_ref, wq_ref, wk_ref, wv_ref,
                q_ref, k_ref, v_ref, sv_ref):
    dn = (((1,), (1,)), ((), ()))
    q_ref[...] = lax.dot_general(qi_ref[...], wq_ref[...], dn,
                                 preferred_element_type=jnp.float32)
    k_ref[...] = lax.dot_general(ki_ref[...], wk_ref[...], dn,
                                 preferred_element_type=jnp.float32)
    v = lax.dot_general(vi_ref[...], wv_ref[...], dn,
                        preferred_element_type=jnp.float32)
    v_ref[...] = v
    part = jnp.sum(v, axis=0, keepdims=True)

    @pl.when(pl.program_id(0) == 0)
    def _init():
        sv_ref[...] = part

    @pl.when(pl.program_id(0) != 0)
    def _acc():
        sv_ref[...] = sv_ref[...] + part


def _project(Qa, Ka, Va, Wq, Wk, Wv):
    blk = pl.BlockSpec((_T, _DM), lambda b: (b, 0))
    wblk = pl.BlockSpec((_DM, _DM), lambda b: (0, 0))
    sd = jax.ShapeDtypeStruct((_S, _DM), jnp.float32)
    return pl.pallas_call(
        _proj3_body,
        grid=(_NB,),
        in_specs=[blk, blk, blk, wblk, wblk, wblk],
        out_specs=[blk, blk, blk, pl.BlockSpec((1, _DM), lambda b: (0, 0))],
        out_shape=[sd, sd, sd, jax.ShapeDtypeStruct((1, _DM), jnp.float32)],
    )(Qa, Ka, Va, Wq, Wk, Wv)


def _sc_gather_body(k_hbm, v_hbm, idx_hbm, ok_hbm, ov_hbm, idx_v, rows_v, sem):
    wid = lax.axis_index("s") * _NC + lax.axis_index("c")
    base = wid * _ROWS_PER_W
    pltpu.sync_copy(idx_hbm.at[pl.ds(base, _ROWS_PER_W)], idx_v)
    pltpu.async_copy(k_hbm.at[idx_v], rows_v, sem).wait()
    pltpu.sync_copy(rows_v, ok_hbm.at[pl.ds(base, _ROWS_PER_W)])
    pltpu.async_copy(v_hbm.at[idx_v], rows_v, sem).wait()
    pltpu.sync_copy(rows_v, ov_hbm.at[pl.ds(base, _ROWS_PER_W)])


def _sc_gather(k, v, ridx):
    mesh = plsc.VectorSubcoreMesh(core_axis_name="c", subcore_axis_name="s")
    run = functools.partial(
        pl.kernel,
        mesh=mesh,
        out_type=[
            jax.ShapeDtypeStruct((_NIDX, _DM), jnp.float32),
            jax.ShapeDtypeStruct((_NIDX, _DM), jnp.float32),
        ],
        scratch_types=[
            pltpu.VMEM((_ROWS_PER_W,), jnp.int32),
            pltpu.VMEM((_ROWS_PER_W, _DM), jnp.float32),
            pltpu.SemaphoreType.DMA,
        ],
    )(_sc_gather_body)
    return run(k, v, ridx)


def _segmats():
    d_idx = lax.broadcasted_iota(jnp.int32, (_DM, _H), 0)
    h_idx = lax.broadcasted_iota(jnp.int32, (_DM, _H), 1)
    seg = (d_idx // _DH == h_idx).astype(jnp.float32)
    d_idx2 = lax.broadcasted_iota(jnp.int32, (_H, _DM), 1)
    h_idx2 = lax.broadcasted_iota(jnp.int32, (_H, _DM), 0)
    segt = (d_idx2 // _DH == h_idx2).astype(jnp.float32)
    return seg, segt


def _seg_blockdiag(n):
    r = lax.broadcasted_iota(jnp.int32, (n * _DM, n * _H), 0)
    c = lax.broadcasted_iota(jnp.int32, (n * _DM, n * _H), 1)
    return ((r // _DM == c // _H)
            & ((r % _DM) // _DH == c % _H)).astype(jnp.float32)


def _segt_blockdiag(n):
    r = lax.broadcasted_iota(jnp.int32, (n * _H, n * _DM), 0)
    c = lax.broadcasted_iota(jnp.int32, (n * _H, n * _DM), 1)
    return ((r // _H == c // _DM)
            & ((c % _DM) // _DH == r % _H)).astype(jnp.float32)


def _attn_body(q_ref, k_ref, v_ref, kr_ref, vr_ref, m_ref, sv_ref, wo_ref,
               o_ref):
    b = pl.program_id(0)
    base = b * _T
    q = q_ref[...]
    seg, segt = _segmats()
    dn_nt = (((1,), (0,)), ((), ()))
    dn_tt = (((1,), (1,)), ((), ()))
    zrow = jnp.zeros((1, _DM), jnp.float32)

    base = pl.multiple_of(base, _T)

    def shifts(ref):
        c = ref[pl.ds(base, _T)]
        pstart = pl.multiple_of(jnp.maximum(base - 8, 0), 8)
        pchunk = ref[pl.ds(pstart, 8)]
        prow = jnp.where(b == 0, zrow, pchunk[7:8])
        nstart = pl.multiple_of(jnp.minimum(base + _T, _S - 8), 8)
        nchunk = ref[pl.ds(nstart, 8)]
        nrow = jnp.where(b == _NB - 1, zrow, nchunk[0:1])
        m1 = jnp.concatenate([prow, c[:-1]], axis=0)
        p1 = jnp.concatenate([c[1:], nrow], axis=0)
        return c, m1, p1

    kc, km1, kp1 = shifts(k_ref)
    vc, vm1, vp1 = shifts(v_ref)
    k0 = k_ref[0:1]
    v0 = v_ref[0:1]
    kr1 = kr_ref[0]
    kr2 = kr_ref[1]
    vr1 = vr_ref[0]
    vr2 = vr_ref[1]

    m6 = jnp.concatenate(
        [q * k0, q * km1, q * kc, q * kp1, q * kr1, q * kr2], axis=1)
    s6 = lax.dot_general(m6, _seg_blockdiag(6), dn_nt,
                         preferred_element_type=jnp.float32) * _SCALE
    e6 = jnp.exp(s6) - 1.0

    rows_g = base + lax.broadcasted_iota(jnp.int32, (_T, 1), 0)
    w0 = e6[:, 0 * _H:1 * _H] * (rows_g != 1).astype(jnp.float32)
    wm1 = e6[:, 1 * _H:2 * _H]
    wc = e6[:, 2 * _H:3 * _H]
    wp1 = e6[:, 3 * _H:4 * _H]
    wr1 = e6[:, 4 * _H:5 * _H]
    wr2 = e6[:, 5 * _H:6 * _H] * m_ref[...]

    den = float(_S) + w0 + wm1 + wc + wp1 + wr1 + wr2
    inv = 1.0 / den

    w7 = jnp.concatenate([w0, wm1, wc, wp1, wr1, wr2, inv], axis=1)
    p7 = lax.dot_general(w7, _segt_blockdiag(7), dn_nt,
                         preferred_element_type=jnp.float32)

    num = (sv_ref[...]
           + p7[:, 0 * _DM:1 * _DM] * v0
           + p7[:, 1 * _DM:2 * _DM] * vm1
           + p7[:, 2 * _DM:3 * _DM] * vc
           + p7[:, 3 * _DM:4 * _DM] * vp1
           + p7[:, 4 * _DM:5 * _DM] * vr1
           + p7[:, 5 * _DM:6 * _DM] * vr2)
    x = num * p7[:, 6 * _DM:7 * _DM]

    o_ref[...] = lax.dot_general(x, wo_ref[...], dn_tt,
                                 preferred_element_type=jnp.float32)

    def global_row(qrow):
        kfull = k_ref[...]
        vfull = v_ref[...]
        sg = lax.dot_general(kfull * qrow, seg, dn_nt,
                             preferred_element_type=jnp.float32) * _SCALE
        sg = sg - jnp.max(sg, axis=0, keepdims=True)
        eg = jnp.exp(sg)
        pg = eg / jnp.sum(eg, axis=0, keepdims=True)
        pb = lax.dot_general(pg, segt, dn_nt,
                             preferred_element_type=jnp.float32)
        og = jnp.sum(pb * vfull, axis=0, keepdims=True)
        return lax.dot_general(og, wo_ref[...], dn_tt,
                               preferred_element_type=jnp.float32)

    @pl.when(b == 0)
    def _g0():
        o_ref[0:1, :] = global_row(q[0:1])

    @pl.when(b == _NB - 1)
    def _g1():
        o_ref[_T - 1:_T, :] = global_row(q[_T - 1:_T])


def _attention(q, k, v, krv, vrv, rmask, sumv, wo):
    full = pl.BlockSpec((_S, _DM), lambda b: (0, 0))
    return pl.pallas_call(
        _attn_body,
        grid=(_NB,),
        in_specs=[
            pl.BlockSpec((_T, _DM), lambda b: (b, 0)),
            full,
            full,
            pl.BlockSpec((2, _T, _DM), lambda b: (0, b, 0)),
            pl.BlockSpec((2, _T, _DM), lambda b: (0, b, 0)),
            pl.BlockSpec((_T, 1), lambda b: (b, 0)),
            pl.BlockSpec((1, _DM), lambda b: (0, 0)),
            pl.BlockSpec((_DM, _DM), lambda b: (0, 0)),
        ],
        out_specs=pl.BlockSpec((_T, _DM), lambda b: (b, 0)),
        out_shape=jax.ShapeDtypeStruct((_S, _DM), jnp.float32),
    )(q, k, v, krv, vrv, rmask, sumv, wo)


def kernel(Q, K, V, Wq, Wk, Wv, Wo, idx_tensor):
    q, k, v, sumv = _project(Q[0], K[0], V[0], Wq, Wk, Wv)

    idx = idx_tensor[0, 0].astype(jnp.int32)
    r1 = jnp.pad(idx[:, 1], (1, 1))
    r2 = jnp.pad(idx[:, 2], (1, 1))
    ridx = jnp.concatenate([r1, r2])
    k_rand, v_rand = _sc_gather(k, v, ridx)

    krv = k_rand.reshape(2, _S, _DM)
    vrv = v_rand.reshape(2, _S, _DM)
    rmask = (r1 != r2).astype(jnp.float32)[:, None]

    out = _attention(q, k, v, krv, vrv, rmask, sumv, Wo)
    return out[None]


# speedup vs baseline: 1.1900x; 1.1900x over previous
import functools

import jax
import jax.numpy as jnp
from jax import lax
from jax.experimental import pallas as pl
from jax.experimental.pallas import tpu as pltpu
from jax.experimental.pallas import tpu_sc as plsc

_S = 2048
_H = 12
_DH = 64
_DM = 768
_NC = 2
_NS = 16
_NW = _NC * _NS
_NIDX = 2 * _S
_ROWS_PER_W = _NIDX // _NW
_T = 512
_NB = _S // _T
_SCALE = 1.0 / 8.0


def _proj3_body(qi_ref, ki_ref, vi_ref, wq_ref, wk_ref, wv_ref,
                q_ref, k_ref, v_ref, sv_ref):
    dn = (((1,), (1,)), ((), ()))
    q_ref[...] = lax.dot_general(qi_ref[...], wq_ref[...], dn,
                                 preferred_element_type=jnp.float32)
    k_ref[...] = lax.dot_general(ki_ref[...], wk_ref[...], dn,
                                 preferred_element_type=jnp.float32)
    v = lax.dot_general(vi_ref[...], wv_ref[...], dn,
                        preferred_element_type=jnp.float32)
    v_ref[...] = v
    part = jnp.sum(v, axis=0, keepdims=True)

    @pl.when(pl.program_id(0) == 0)
    def _init():
        sv_ref[...] = part

    @pl.when(pl.program_id(0) != 0)
    def _acc():
        sv_ref[...] = sv_ref[...] + part


def _project(Qa, Ka, Va, Wq, Wk, Wv):
    blk = pl.BlockSpec((_T, _DM), lambda b: (b, 0))
    wblk = pl.BlockSpec((_DM, _DM), lambda b: (0, 0))
    sd = jax.ShapeDtypeStruct((_S, _DM), jnp.float32)
    return pl.pallas_call(
        _proj3_body,
        grid=(_NB,),
        in_specs=[blk, blk, blk, wblk, wblk, wblk],
        out_specs=[blk, blk, blk, pl.BlockSpec((1, _DM), lambda b: (0, 0))],
        out_shape=[sd, sd, sd, jax.ShapeDtypeStruct((1, _DM), jnp.float32)],
    )(Qa, Ka, Va, Wq, Wk, Wv)


def _sc_gather_body(k_hbm, v_hbm, idx_hbm, ok_hbm, ov_hbm, idx_v, rows_v, sem):
    wid = lax.axis_index("s") * _NC + lax.axis_index("c")
    base = wid * _ROWS_PER_W
    pltpu.sync_copy(idx_hbm.at[pl.ds(base, _ROWS_PER_W)], idx_v)
    pltpu.async_copy(k_hbm.at[idx_v], rows_v, sem).wait()
    pltpu.sync_copy(rows_v, ok_hbm.at[pl.ds(base, _ROWS_PER_W)])
    pltpu.async_copy(v_hbm.at[idx_v], rows_v, sem).wait()
    pltpu.sync_copy(rows_v, ov_hbm.at[pl.ds(base, _ROWS_PER_W)])


def _sc_gather(k, v, ridx):
    mesh = plsc.VectorSubcoreMesh(core_axis_name="c", subcore_axis_name="s")
    run = functools.partial(
        pl.kernel,
        mesh=mesh,
        out_type=[
            jax.ShapeDtypeStruct((_NIDX, _DM), jnp.float32),
            jax.ShapeDtypeStruct((_NIDX, _DM), jnp.float32),
        ],
        scratch_types=[
            pltpu.VMEM((_ROWS_PER_W,), jnp.int32),
            pltpu.VMEM((_ROWS_PER_W, _DM), jnp.float32),
            pltpu.SemaphoreType.DMA,
        ],
    )(_sc_gather_body)
    return run(k, v, ridx)


def _segmats():
    d_idx = lax.broadcasted_iota(jnp.int32, (_DM, _H), 0)
    h_idx = lax.broadcasted_iota(jnp.int32, (_DM, _H), 1)
    seg = (d_idx // _DH == h_idx).astype(jnp.float32)
    d_idx2 = lax.broadcasted_iota(jnp.int32, (_H, _DM), 1)
    h_idx2 = lax.broadcasted_iota(jnp.int32, (_H, _DM), 0)
    segt = (d_idx2 // _DH == h_idx2).astype(jnp.float32)
    return seg, segt


def _seg_blockdiag(n):
    r = lax.broadcasted_iota(jnp.int32, (n * _DM, n * _H), 0)
    c = lax.broadcasted_iota(jnp.int32, (n * _DM, n * _H), 1)
    return ((r // _DM == c // _H)
            & ((r % _DM) // _DH == c % _H)).astype(jnp.float32)


def _segt_blockdiag(n):
    r = lax.broadcasted_iota(jnp.int32, (n * _H, n * _DM), 0)
    c = lax.broadcasted_iota(jnp.int32, (n * _H, n * _DM), 1)
    return ((r // _H == c // _DM)
            & ((c % _DM) // _DH == r % _H)).astype(jnp.float32)


def _attn_a_body(q_ref, k_ref, v_ref, sv_ref, wo_ref,
                 num_ref, den_ref, g_ref):
    b = pl.program_id(0)
    base = b * _T
    q = q_ref[...]
    seg, segt = _segmats()
    dn_nt = (((1,), (0,)), ((), ()))
    dn_tt = (((1,), (1,)), ((), ()))
    zrow = jnp.zeros((1, _DM), jnp.float32)

    base = pl.multiple_of(base, _T)

    def shifts(ref):
        c = ref[pl.ds(base, _T)]
        pstart = pl.multiple_of(jnp.maximum(base - 8, 0), 8)
        pchunk = ref[pl.ds(pstart, 8)]
        prow = jnp.where(b == 0, zrow, pchunk[7:8])
        nstart = pl.multiple_of(jnp.minimum(base + _T, _S - 8), 8)
        nchunk = ref[pl.ds(nstart, 8)]
        nrow = jnp.where(b == _NB - 1, zrow, nchunk[0:1])
        m1 = jnp.concatenate([prow, c[:-1]], axis=0)
        p1 = jnp.concatenate([c[1:], nrow], axis=0)
        return c, m1, p1

    kc, km1, kp1 = shifts(k_ref)
    vc, vm1, vp1 = shifts(v_ref)
    k0 = k_ref[0:1]
    v0 = v_ref[0:1]

    def wexp(kk):
        s = lax.dot_general(q * kk, seg, dn_nt,
                            preferred_element_type=jnp.float32) * _SCALE
        return jnp.exp(s) - 1.0

    rows_g = base + lax.broadcasted_iota(jnp.int32, (_T, 1), 0)
    w0 = wexp(k0) * (rows_g != 1).astype(jnp.float32)
    wm1 = wexp(km1)
    wc = wexp(kc)
    wp1 = wexp(kp1)

    den_ref[...] = float(_S) + w0 + wm1 + wc + wp1

    def bcast(w):
        return lax.dot_general(w, segt, dn_nt,
                               preferred_element_type=jnp.float32)

    num_ref[...] = (sv_ref[...] + bcast(w0) * v0 + bcast(wm1) * vm1
                    + bcast(wc) * vc + bcast(wp1) * vp1)

    def global_row(qrow):
        kfull = k_ref[...]
        vfull = v_ref[...]
        sg = lax.dot_general(kfull * qrow, seg, dn_nt,
                             preferred_element_type=jnp.float32) * _SCALE
        sg = sg - jnp.max(sg, axis=0, keepdims=True)
        eg = jnp.exp(sg)
        pg = eg / jnp.sum(eg, axis=0, keepdims=True)
        pb = lax.dot_general(pg, segt, dn_nt,
                             preferred_element_type=jnp.float32)
        og = jnp.sum(pb * vfull, axis=0, keepdims=True)
        return lax.dot_general(og, wo_ref[...], dn_tt,
                               preferred_element_type=jnp.float32)

    @pl.when(b == 0)
    def _g0():
        g_ref[0:1, :] = global_row(q[0:1])

    @pl.when(b == _NB - 1)
    def _g1():
        g_ref[1:2, :] = global_row(q[_T - 1:_T])


def _attn_a(q, k, v, sumv, wo):
    full = pl.BlockSpec((_S, _DM), lambda b: (0, 0))
    return pl.pallas_call(
        _attn_a_body,
        grid=(_NB,),
        in_specs=[
            pl.BlockSpec((_T, _DM), lambda b: (b, 0)),
            full,
            full,
            pl.BlockSpec((1, _DM), lambda b: (0, 0)),
            pl.BlockSpec((_DM, _DM), lambda b: (0, 0)),
        ],
        out_specs=[
            pl.BlockSpec((_T, _DM), lambda b: (b, 0)),
            pl.BlockSpec((_T, _H), lambda b: (b, 0)),
            pl.BlockSpec((8, _DM), lambda b: (0, 0)),
        ],
        out_shape=[
            jax.ShapeDtypeStruct((_S, _DM), jnp.float32),
            jax.ShapeDtypeStruct((_S, _H), jnp.float32),
            jax.ShapeDtypeStruct((8, _DM), jnp.float32),
        ],
    )(q, k, v, sumv, wo)


def _attn_b_body(q_ref, kr_ref, vr_ref, m_ref, num_ref, den_ref, g_ref,
                 wo_ref, o_ref):
    b = pl.program_id(0)
    q = q_ref[...]
    seg, segt = _segmats()
    dn_nt = (((1,), (0,)), ((), ()))
    dn_tt = (((1,), (1,)), ((), ()))

    kr1 = kr_ref[0]
    kr2 = kr_ref[1]
    vr1 = vr_ref[0]
    vr2 = vr_ref[1]

    def wexp(kk):
        s = lax.dot_general(q * kk, seg, dn_nt,
                            preferred_element_type=jnp.float32) * _SCALE
        return jnp.exp(s) - 1.0

    wr1 = wexp(kr1)
    wr2 = wexp(kr2) * m_ref[...]
    inv = 1.0 / (den_ref[...] + wr1 + wr2)

    def bcast(w):
        return lax.dot_general(w, segt, dn_nt,
                               preferred_element_type=jnp.float32)

    num = num_ref[...] + bcast(wr1) * vr1 + bcast(wr2) * vr2
    x = num * bcast(inv)
    o_ref[...] = lax.dot_general(x, wo_ref[...], dn_tt,
                                 preferred_element_type=jnp.float32)

    @pl.when(b == 0)
    def _g0():
        o_ref[0:1, :] = g_ref[0:1, :]

    @pl.when(b == _NB - 1)
    def _g1():
        o_ref[_T - 1:_T, :] = g_ref[1:2, :]


def _attn_b(q, krv, vrv, rmask, numa, dena, gout, wo):
    return pl.pallas_call(
        _attn_b_body,
        grid=(_NB,),
        in_specs=[
            pl.BlockSpec((_T, _DM), lambda b: (b, 0)),
            pl.BlockSpec((2, _T, _DM), lambda b: (0, b, 0)),
            pl.BlockSpec((2, _T, _DM), lambda b: (0, b, 0)),
            pl.BlockSpec((_T, 1), lambda b: (b, 0)),
            pl.BlockSpec((_T, _DM), lambda b: (b, 0)),
            pl.BlockSpec((_T, _H), lambda b: (b, 0)),
            pl.BlockSpec((8, _DM), lambda b: (0, 0)),
            pl.BlockSpec((_DM, _DM), lambda b: (0, 0)),
        ],
        out_specs=pl.BlockSpec((_T, _DM), lambda b: (b, 0)),
        out_shape=jax.ShapeDtypeStruct((_S, _DM), jnp.float32),
    )(q, krv, vrv, rmask, numa, dena, gout, wo)


def kernel(Q, K, V, Wq, Wk, Wv, Wo, idx_tensor):
    q, k, v, sumv = _project(Q[0], K[0], V[0], Wq, Wk, Wv)

    idx = idx_tensor[0, 0].astype(jnp.int32)
    r1 = jnp.pad(idx[:, 1], (1, 1))
    r2 = jnp.pad(idx[:, 2], (1, 1))
    ridx = jnp.concatenate([r1, r2])
    k_rand, v_rand = _sc_gather(k, v, ridx)

    krv = k_rand.reshape(2, _S, _DM)
    vrv = v_rand.reshape(2, _S, _DM)
    rmask = (r1 != r2).astype(jnp.float32)[:, None]

    numa, dena, gout = _attn_a(q, k, v, sumv, Wo)
    out = _attn_b(q, krv, vrv, rmask, numa, dena, gout, Wo)
    return out[None]
